# baseline (speedup 1.0000x reference)
import jax
import jax.numpy as jnp
from jax import lax
from jax.experimental import pallas as pl
from jax.experimental.pallas import tpu as pltpu
from jax.experimental.pallas import tpu_sc as plsc

N = 10000
E = 320000
D = 128
H = 8
DH = 16
N_PAD = 10240
ROW = D + DH

NC = 2
NS = 16
NW = NC * NS
E_PER_W = E // NW
C = 40
CHUNKS = E_PER_W // C
ROWS_PER_TILE = N_PAD // NS

LANE_OF = (0, 8, 4, 12, 2, 10, 6, 14)

_GDN = lax.GatherDimensionNumbers(
    offset_dims=(), collapsed_slice_dims=(0,), start_index_map=(0,))


def _shuf(x, idx):
    return lax.gather(x, idx[:, None], _GDN, (1,),
                      mode=lax.GatherScatterMode.PROMISE_IN_BOUNDS)


def _edge_kernel(q_hbm, kvx_hbm, kvr_hbm, idx3_hbm, out_hbm,
                 ix, wrow, qv, kv, acc, sems):
    cid = lax.axis_index("c")
    sid = lax.axis_index("s")
    wid = cid * NS + sid
    gbase = wid * CHUNKS

    def _zrow(i, _):
        for j in range(ROW // 16):
            wrow[i, pl.ds(j * 16, 16)] = jnp.zeros((16,), jnp.float32)
        return 0
    lax.fori_loop(0, C, _zrow, 0)
    row0 = sid * ROWS_PER_TILE
    for b in range(ROWS_PER_TILE // C):
        pltpu.sync_copy(wrow, acc.at[pl.ds(row0 + b * C, C)])
    plsc.subcore_barrier()

    lanes = lax.iota(jnp.int32, 16)
    px = {h: lanes ^ h for h in (8, 4, 2, 1)}
    msk = {h: (lanes & h) == 0 for h in (8, 4, 2)}
    bcast = [jnp.full((16,), LANE_OF[h], jnp.int32) for h in range(H)]

    def _merge(a, b, half):
        a2 = a + _shuf(a, px[half])
        b2 = b + _shuf(b, px[half])
        return jnp.where(msk[half], a2, _shuf(b2, px[half]))

    def _load_idx(c, p):
        pltpu.async_copy(idx3_hbm.at[gbase + c], ix.at[p], sems.at[2]).wait()

    def _issue(p):
        pltpu.async_copy(q_hbm.at[ix.at[p, 1]], qv.at[p], sems.at[p])
        pltpu.async_copy(kvx_hbm.at[ix.at[p, 0]], kv.at[p], sems.at[p])

    def _wait_gathers(p):
        pltpu.make_async_copy(q_hbm.at[ix.at[p, 1]], qv.at[p],
                              sems.at[p]).wait()
        pltpu.make_async_copy(kvx_hbm.at[ix.at[p, 0]], kv.at[p],
                              sems.at[p]).wait()

    def _compute_scatter(p):
        @plsc.parallel_loop(0, C, 1, unroll=4)
        def _edge(e):
            u = [qv[p, e, pl.ds(h * 16, 16)] * kv[p, e, pl.ds(h * 16, 16)]
                 for h in range(H)]
            v1 = [_merge(u[2 * i], u[2 * i + 1], 8) for i in range(4)]
            v2 = [_merge(v1[2 * i], v1[2 * i + 1], 4) for i in range(2)]
            r = _merge(v2[0], v2[1], 2)
            r = r + _shuf(r, px[1])
            pv = jnp.exp(r)
            for h in range(H):
                ph = _shuf(pv, bcast[h])
                wrow[e, pl.ds(h * 16, 16)] = ph * kv[p, e, pl.ds(D + h * 16, 16)]
            wrow[e, pl.ds(D, 16)] = pv
        pltpu.sync_copy(wrow, acc.at[ix.at[p, 1]], add=True)

    def _body(i, _):
        c0 = 2 * i
        _wait_gathers(0)
        ga = pltpu.async_copy(kvr_hbm.at[ix.at[0, 2]], kv.at[0], sems.at[0],
                              add=True)
        _load_idx(c0 + 1, 1)
        _issue(1)
        ga.wait()
        _compute_scatter(0)
        _wait_gathers(1)
        gb = pltpu.async_copy(kvr_hbm.at[ix.at[1, 2]], kv.at[1], sems.at[1],
                              add=True)
        c2 = lax.min(c0 + 2, CHUNKS - 1)
        _load_idx(c2, 0)
        _issue(0)
        gb.wait()
        _compute_scatter(1)
        return 0

    _load_idx(0, 0)
    _issue(0)
    lax.fori_loop(0, CHUNKS // 2, _body, 0)
    _wait_gathers(0)

    plsc.subcore_barrier()
    for b in range(ROWS_PER_TILE // C):
        r0 = row0 + b * C
        pltpu.sync_copy(acc.at[pl.ds(r0, C)], wrow)
        pltpu.sync_copy(wrow, out_hbm.at[cid, pl.ds(r0, C)])


_edge_pass = pl.kernel(
    _edge_kernel,
    out_type=jax.ShapeDtypeStruct((NC, N_PAD, ROW), jnp.float32),
    mesh=plsc.VectorSubcoreMesh(core_axis_name="c", subcore_axis_name="s"),
    compiler_params=pltpu.CompilerParams(needs_layout_passes=False,
                                         use_tc_tiling_on_sc=False),
    scratch_types=[
        pltpu.VMEM((2, 3, C), jnp.int32),
        pltpu.VMEM((C, ROW), jnp.float32),
        pltpu.VMEM((2, C, D), jnp.float32),
        pltpu.VMEM((2, C, 2 * D), jnp.float32),
        pltpu.VMEM_SHARED((N_PAD, ROW), jnp.float32),
        pltpu.SemaphoreType.DMA((3,)),
    ],
)


def _proj_body(x_ref, w_ref, q_ref, kv_ref):
    y = jnp.dot(x_ref[...], w_ref[...],
                preferred_element_type=jnp.float32,
                precision=lax.Precision.HIGHEST)
    q_ref[...] = y[:, :D]
    kv_ref[...] = y[:, D:]


def _project(x, w_cat):
    bn = 400
    return pl.pallas_call(
        _proj_body,
        grid=(N // bn,),
        in_specs=[pl.BlockSpec((bn, D), lambda i: (i, 0)),
                  pl.BlockSpec((D, 3 * D), lambda i: (0, 0))],
        out_specs=[pl.BlockSpec((bn, D), lambda i: (i, 0)),
                   pl.BlockSpec((bn, 2 * D), lambda i: (i, 0))],
        out_shape=[jax.ShapeDtypeStruct((N, D), jnp.float32),
                   jax.ShapeDtypeStruct((N, 2 * D), jnp.float32)],
    )(x, w_cat)


def _rel_body(r_ref, w_ref, o_ref):
    o_ref[...] = jnp.dot(r_ref[...], w_ref[...],
                         preferred_element_type=jnp.float32,
                         precision=lax.Precision.HIGHEST)


def _project_rel(rel_emb, w_kv):
    nr = rel_emb.shape[0]
    return pl.pallas_call(
        _rel_body,
        out_shape=jax.ShapeDtypeStruct((nr, 2 * D), jnp.float32),
    )(rel_emb, w_kv)


def _fin_body(a_ref, ex_ref, wo_ref, o_ref):
    a = a_ref[0] + a_ref[1]
    den = jnp.dot(a, ex_ref[...], preferred_element_type=jnp.float32,
                  precision=lax.Precision.HIGHEST)
    agg = a[:, :D] / (den + 1e-16)
    o_ref[...] = jnp.dot(agg, wo_ref[...], preferred_element_type=jnp.float32,
                         precision=lax.Precision.HIGHEST)


def _finalize(part, expand, w_o):
    bn = 400
    return pl.pallas_call(
        _fin_body,
        grid=(N // bn,),
        in_specs=[pl.BlockSpec((NC, bn, ROW), lambda i: (0, i, 0)),
                  pl.BlockSpec((ROW, D), lambda i: (0, 0)),
                  pl.BlockSpec((D, D), lambda i: (0, 0))],
        out_specs=pl.BlockSpec((bn, D), lambda i: (i, 0)),
        out_shape=jax.ShapeDtypeStruct((N, D), jnp.float32),
    )(part, expand, w_o)


def kernel(x, edge_index, edge_type, W_q, W_k, W_v, W_o, rel_emb):
    src = edge_index[0].astype(jnp.int32)
    dst = edge_index[1].astype(jnp.int32)
    et = edge_type.astype(jnp.int32)

    w_cat = jnp.concatenate([W_q * (1.0 / jnp.sqrt(jnp.float32(DH))),
                             W_k, W_v], axis=1)
    q_all, kv_x = _project(x, w_cat)
    kv_r = _project_rel(rel_emb, w_cat[:, D:])

    idx3 = jnp.stack([src, dst, et]).reshape(3, NW * CHUNKS, C)
    idx3 = idx3.transpose(1, 0, 2)

    part = _edge_pass(q_all, kv_x, kv_r, idx3)

    expand = jnp.zeros((ROW, D), jnp.float32)
    for h in range(H):
        expand = expand.at[D + LANE_OF[h], h * DH:(h + 1) * DH].set(1.0)

    return _finalize(part, expand, W_o)


# speedup vs baseline: 1.2960x; 1.2960x over previous
import jax
import jax.numpy as jnp
from jax import lax
from jax.experimental import pallas as pl
from jax.experimental.pallas import tpu as pltpu
from jax.experimental.pallas import tpu_sc as plsc

N = 10000
E = 320000
D = 128
H = 8
DH = 16
N_PAD = 10240
ROW = D + DH

NC = 2
NS = 16
NW = NC * NS
E_PER_W = E // NW
C = 40
CHUNKS = E_PER_W // C
ROWS_PER_TILE = N_PAD // NS

LANE_OF = (0, 8, 4, 12, 2, 10, 6, 14)

_GDN = lax.GatherDimensionNumbers(
    offset_dims=(), collapsed_slice_dims=(0,), start_index_map=(0,))


def _shuf(x, idx):
    return lax.gather(x, idx[:, None], _GDN, (1,),
                      mode=lax.GatherScatterMode.PROMISE_IN_BOUNDS)


def _edge_kernel(q_hbm, kvx_hbm, kvr_hbm, idx3_hbm, out_hbm,
                 ix, wrow, qv, kv, acc, sems):
    cid = lax.axis_index("c")
    sid = lax.axis_index("s")
    wid = cid * NS + sid
    gbase = wid * CHUNKS

    def _zrow(i, _):
        for j in range(ROW // 16):
            wrow[i, pl.ds(j * 16, 16)] = jnp.zeros((16,), jnp.float32)
        return 0
    lax.fori_loop(0, C, _zrow, 0)
    row0 = sid * ROWS_PER_TILE
    for b in range(ROWS_PER_TILE // C):
        pltpu.sync_copy(wrow, acc.at[pl.ds(row0 + b * C, C)])
    plsc.subcore_barrier()

    lanes = lax.iota(jnp.int32, 16)
    px = {h: lanes ^ h for h in (8, 4, 2, 1)}
    msk = {h: (lanes & h) == 0 for h in (8, 4, 2)}
    bcast = [jnp.full((16,), LANE_OF[h], jnp.int32) for h in range(H)]

    def _merge(a, b, half):
        a2 = a + _shuf(a, px[half])
        b2 = b + _shuf(b, px[half])
        return jnp.where(msk[half], a2, _shuf(b2, px[half]))

    def _load_idx(c, p):
        pltpu.async_copy(idx3_hbm.at[gbase + c], ix.at[p], sems.at[2]).wait()

    def _issue(p):
        pltpu.async_copy(q_hbm.at[ix.at[p, 1]], qv.at[p], sems.at[p])
        pltpu.async_copy(kvx_hbm.at[ix.at[p, 0]], kv.at[p], sems.at[p])

    def _wait_gathers(p):
        pltpu.make_async_copy(q_hbm.at[ix.at[p, 1]], qv.at[p],
                              sems.at[p]).wait()
        pltpu.make_async_copy(kvx_hbm.at[ix.at[p, 0]], kv.at[p],
                              sems.at[p]).wait()

    def _compute_scatter(p):
        @plsc.parallel_loop(0, C, 1, unroll=2)
        def _edge(e):
            u = [qv[p, e, pl.ds(h * 16, 16)] * kv[p, e, pl.ds(h * 16, 16)]
                 for h in range(H)]
            v1 = [_merge(u[2 * i], u[2 * i + 1], 8) for i in range(4)]
            v2 = [_merge(v1[2 * i], v1[2 * i + 1], 4) for i in range(2)]
            r = _merge(v2[0], v2[1], 2)
            r = r + _shuf(r, px[1])
            pv = jnp.exp(r)
            for h in range(H):
                ph = _shuf(pv, bcast[h])
                wrow[e, pl.ds(h * 16, 16)] = ph * kv[p, e, pl.ds(D + h * 16, 16)]
            wrow[e, pl.ds(D, 16)] = pv
        pltpu.sync_copy(wrow, acc.at[ix.at[p, 1]], add=True)

    def _body(i, _):
        c0 = 2 * i
        _wait_gathers(0)
        ga = pltpu.async_copy(kvr_hbm.at[ix.at[0, 2]], kv.at[0], sems.at[0],
                              add=True)
        _load_idx(c0 + 1, 1)
        _issue(1)
        ga.wait()
        _compute_scatter(0)
        _wait_gathers(1)
        gb = pltpu.async_copy(kvr_hbm.at[ix.at[1, 2]], kv.at[1], sems.at[1],
                              add=True)
        c2 = lax.min(c0 + 2, CHUNKS - 1)
        _load_idx(c2, 0)
        _issue(0)
        gb.wait()
        _compute_scatter(1)
        return 0

    _load_idx(0, 0)
    _issue(0)
    lax.fori_loop(0, CHUNKS // 2, _body, 0)
    _wait_gathers(0)

    plsc.subcore_barrier()
    for b in range(ROWS_PER_TILE // C):
        r0 = row0 + b * C
        pltpu.sync_copy(acc.at[pl.ds(r0, C)], wrow)
        pltpu.sync_copy(wrow, out_hbm.at[cid, pl.ds(r0, C)])


_edge_pass = pl.kernel(
    _edge_kernel,
    out_type=jax.ShapeDtypeStruct((NC, N_PAD, ROW), jnp.float32),
    mesh=plsc.VectorSubcoreMesh(core_axis_name="c", subcore_axis_name="s"),
    compiler_params=pltpu.CompilerParams(needs_layout_passes=False,
                                         use_tc_tiling_on_sc=False),
    scratch_types=[
        pltpu.VMEM((2, 3, C), jnp.int32),
        pltpu.VMEM((C, ROW), jnp.float32),
        pltpu.VMEM((2, C, D), jnp.float32),
        pltpu.VMEM((2, C, 2 * D), jnp.float32),
        pltpu.VMEM_SHARED((N_PAD, ROW), jnp.float32),
        pltpu.SemaphoreType.DMA((3,)),
    ],
)


def _proj_body(x_ref, w_ref, q_ref, kv_ref):
    y = jnp.dot(x_ref[...], w_ref[...],
                preferred_element_type=jnp.float32,
                precision=lax.Precision.HIGHEST)
    q_ref[...] = y[:, :D]
    kv_ref[...] = y[:, D:]


def _project(x, w_cat):
    bn = 400
    return pl.pallas_call(
        _proj_body,
        grid=(N // bn,),
        in_specs=[pl.BlockSpec((bn, D), lambda i: (i, 0)),
                  pl.BlockSpec((D, 3 * D), lambda i: (0, 0))],
        out_specs=[pl.BlockSpec((bn, D), lambda i: (i, 0)),
                   pl.BlockSpec((bn, 2 * D), lambda i: (i, 0))],
        out_shape=[jax.ShapeDtypeStruct((N, D), jnp.float32),
                   jax.ShapeDtypeStruct((N, 2 * D), jnp.float32)],
    )(x, w_cat)


def _rel_body(r_ref, w_ref, o_ref):
    o_ref[...] = jnp.dot(r_ref[...], w_ref[...],
                         preferred_element_type=jnp.float32,
                         precision=lax.Precision.HIGHEST)


def _project_rel(rel_emb, w_kv):
    nr = rel_emb.shape[0]
    return pl.pallas_call(
        _rel_body,
        out_shape=jax.ShapeDtypeStruct((nr, 2 * D), jnp.float32),
    )(rel_emb, w_kv)


def _fin_body(a_ref, ex_ref, wo_ref, o_ref):
    a = a_ref[0] + a_ref[1]
    den = jnp.dot(a, ex_ref[...], preferred_element_type=jnp.float32,
                  precision=lax.Precision.HIGHEST)
    agg = a[:, :D] / (den + 1e-16)
    o_ref[...] = jnp.dot(agg, wo_ref[...], preferred_element_type=jnp.float32,
                         precision=lax.Precision.HIGHEST)


def _finalize(part, expand, w_o):
    bn = 400
    return pl.pallas_call(
        _fin_body,
        grid=(N // bn,),
        in_specs=[pl.BlockSpec((NC, bn, ROW), lambda i: (0, i, 0)),
                  pl.BlockSpec((ROW, D), lambda i: (0, 0)),
                  pl.BlockSpec((D, D), lambda i: (0, 0))],
        out_specs=pl.BlockSpec((bn, D), lambda i: (i, 0)),
        out_shape=jax.ShapeDtypeStruct((N, D), jnp.float32),
    )(part, expand, w_o)


def kernel(x, edge_index, edge_type, W_q, W_k, W_v, W_o, rel_emb):
    src = edge_index[0].astype(jnp.int32)
    dst = edge_index[1].astype(jnp.int32)
    et = edge_type.astype(jnp.int32)

    w_cat = jnp.concatenate([W_q * (1.0 / jnp.sqrt(jnp.float32(DH))),
                             W_k, W_v], axis=1)
    q_all, kv_x = _project(x, w_cat)
    kv_r = _project_rel(rel_emb, w_cat[:, D:])

    idx3 = jnp.stack([src, dst, et]).reshape(3, NW * CHUNKS, C)
    idx3 = idx3.transpose(1, 0, 2)

    part = _edge_pass(q_all, kv_x, kv_r, idx3)

    expand = jnp.zeros((ROW, D), jnp.float32)
    for h in range(H):
        expand = expand.at[D + LANE_OF[h], h * DH:(h + 1) * DH].set(1.0)

    return _finalize(part, expand, W_o)


# speedup vs baseline: 1.4472x; 1.1167x over previous
import jax
import jax.numpy as jnp
from jax import lax
from jax.experimental import pallas as pl
from jax.experimental.pallas import tpu as pltpu
from jax.experimental.pallas import tpu_sc as plsc

N = 10000
E = 320000
D = 128
H = 8
DH = 16
N_ACC = 10080
ROW = D + DH
R = 200

NC = 2
NS = 16
NW = NC * NS
E_PER_W = E // NW
C = 40
CHUNKS = E_PER_W // C
ROWS_PER_TILE = N_ACC // NS
WBR = 30

LANE_OF = (0, 8, 4, 12, 2, 10, 6, 14)

_GDN = lax.GatherDimensionNumbers(
    offset_dims=(), collapsed_slice_dims=(0,), start_index_map=(0,))


def _shuf(x, idx):
    return lax.gather(x, idx[:, None], _GDN, (1,),
                      mode=lax.GatherScatterMode.PROMISE_IN_BOUNDS)


def _edge_kernel(q_hbm, kvx_hbm, kvr_hbm, idx3_hbm, out_hbm,
                 ix, wrow, qv, kv, acc, kvr_spm, sems):
    cid = lax.axis_index("c")
    sid = lax.axis_index("s")
    wid = cid * NS + sid
    gbase = wid * CHUNKS

    @pl.when(sid == 0)
    def _stage_rel():
        pltpu.sync_copy(kvr_hbm, kvr_spm)

    def _zrow(i, _):
        for j in range(ROW // 16):
            wrow[i, pl.ds(j * 16, 16)] = jnp.zeros((16,), jnp.float32)
        return 0
    lax.fori_loop(0, C, _zrow, 0)
    row0 = sid * ROWS_PER_TILE
    for b in range(ROWS_PER_TILE // WBR):
        pltpu.sync_copy(wrow.at[pl.ds(0, WBR)],
                        acc.at[pl.ds(row0 + b * WBR, WBR)])
    plsc.subcore_barrier()

    lanes = lax.iota(jnp.int32, 16)
    px = {h: lanes ^ h for h in (8, 4, 2, 1)}
    msk = {h: (lanes & h) == 0 for h in (8, 4, 2)}
    bcast = [jnp.full((16,), LANE_OF[h], jnp.int32) for h in range(H)]

    def _merge(a, b, half):
        a2 = a + _shuf(a, px[half])
        b2 = b + _shuf(b, px[half])
        return jnp.where(msk[half], a2, _shuf(b2, px[half]))

    def _load_idx(c, p):
        pltpu.async_copy(idx3_hbm.at[gbase + c], ix.at[p], sems.at[2]).wait()

    def _issue(p):
        pltpu.async_copy(q_hbm.at[ix.at[p, 1]], qv.at[p], sems.at[p])
        pltpu.async_copy(kvx_hbm.at[ix.at[p, 0]], kv.at[p], sems.at[p])

    def _wait_gathers(p):
        pltpu.make_async_copy(q_hbm.at[ix.at[p, 1]], qv.at[p],
                              sems.at[p]).wait()
        pltpu.make_async_copy(kvx_hbm.at[ix.at[p, 0]], kv.at[p],
                              sems.at[p]).wait()

    def _compute_scatter(p):
        @plsc.parallel_loop(0, C, 1, unroll=2)
        def _edge(e):
            u = [qv[p, e, pl.ds(h * 16, 16)] * kv[p, e, pl.ds(h * 16, 16)]
                 for h in range(H)]
            v1 = [_merge(u[2 * i], u[2 * i + 1], 8) for i in range(4)]
            v2 = [_merge(v1[2 * i], v1[2 * i + 1], 4) for i in range(2)]
            r = _merge(v2[0], v2[1], 2)
            r = r + _shuf(r, px[1])
            pv = jnp.exp(r)
            for h in range(H):
                ph = _shuf(pv, bcast[h])
                wrow[e, pl.ds(h * 16, 16)] = ph * kv[p, e, pl.ds(D + h * 16, 16)]
            wrow[e, pl.ds(D, 16)] = pv
        pltpu.async_copy(wrow, acc.at[ix.at[p, 1]], sems.at[3], add=True)

    def _wait_scatter():
        pltpu.make_async_copy(wrow, acc.at[ix.at[0, 1]], sems.at[3]).wait()

    def _body(i, _):
        c0 = 2 * i
        _wait_gathers(0)
        ga = pltpu.async_copy(kvr_spm.at[ix.at[0, 2]], kv.at[0], sems.at[0],
                              add=True)
        _wait_scatter()
        _load_idx(c0 + 1, 1)
        _issue(1)
        ga.wait()
        _compute_scatter(0)
        _wait_gathers(1)
        gb = pltpu.async_copy(kvr_spm.at[ix.at[1, 2]], kv.at[1], sems.at[1],
                              add=True)
        _wait_scatter()
        c2 = lax.min(c0 + 2, CHUNKS - 1)
        _load_idx(c2, 0)
        _issue(0)
        gb.wait()
        _compute_scatter(1)
        return 0

    _load_idx(0, 0)
    _issue(0)
    pltpu.async_copy(wrow, acc.at[ix.at[0, 1]], sems.at[3], add=True)
    lax.fori_loop(0, CHUNKS // 2, _body, 0)
    _wait_gathers(0)
    _wait_scatter()

    plsc.subcore_barrier()
    for b in range(ROWS_PER_TILE // WBR):
        r0 = row0 + b * WBR
        pltpu.sync_copy(acc.at[pl.ds(r0, WBR)], wrow.at[pl.ds(0, WBR)])
        pltpu.sync_copy(wrow.at[pl.ds(0, WBR)],
                        out_hbm.at[cid, pl.ds(r0, WBR)])


_edge_pass = pl.kernel(
    _edge_kernel,
    out_type=jax.ShapeDtypeStruct((NC, N_ACC, ROW), jnp.float32),
    mesh=plsc.VectorSubcoreMesh(core_axis_name="c", subcore_axis_name="s"),
    compiler_params=pltpu.CompilerParams(needs_layout_passes=False,
                                         use_tc_tiling_on_sc=False),
    scratch_types=[
        pltpu.VMEM((2, 3, C), jnp.int32),
        pltpu.VMEM((C, ROW), jnp.float32),
        pltpu.VMEM((2, C, D), jnp.float32),
        pltpu.VMEM((2, C, 2 * D), jnp.float32),
        pltpu.VMEM_SHARED((N_ACC, ROW), jnp.float32),
        pltpu.VMEM_SHARED((R, 2 * D), jnp.float32),
        pltpu.SemaphoreType.DMA((4,)),
    ],
)


def _proj_body(x_ref, w_ref, q_ref, kv_ref):
    y = jnp.dot(x_ref[...], w_ref[...],
                preferred_element_type=jnp.float32,
                precision=lax.Precision.HIGHEST)
    q_ref[...] = y[:, :D]
    kv_ref[...] = y[:, D:]


def _project(x, w_cat):
    bn = 400
    return pl.pallas_call(
        _proj_body,
        grid=(N // bn,),
        in_specs=[pl.BlockSpec((bn, D), lambda i: (i, 0)),
                  pl.BlockSpec((D, 3 * D), lambda i: (0, 0))],
        out_specs=[pl.BlockSpec((bn, D), lambda i: (i, 0)),
                   pl.BlockSpec((bn, 2 * D), lambda i: (i, 0))],
        out_shape=[jax.ShapeDtypeStruct((N, D), jnp.float32),
                   jax.ShapeDtypeStruct((N, 2 * D), jnp.float32)],
    )(x, w_cat)


def _rel_body(r_ref, w_ref, o_ref):
    o_ref[...] = jnp.dot(r_ref[...], w_ref[...],
                         preferred_element_type=jnp.float32,
                         precision=lax.Precision.HIGHEST)


def _project_rel(rel_emb, w_kv):
    nr = rel_emb.shape[0]
    return pl.pallas_call(
        _rel_body,
        out_shape=jax.ShapeDtypeStruct((nr, 2 * D), jnp.float32),
    )(rel_emb, w_kv)


def _fin_body(a_ref, ex_ref, wo_ref, o_ref):
    a = a_ref[0] + a_ref[1]
    den = jnp.dot(a, ex_ref[...], preferred_element_type=jnp.float32,
                  precision=lax.Precision.HIGHEST)
    agg = a[:, :D] / (den + 1e-16)
    o_ref[...] = jnp.dot(agg, wo_ref[...], preferred_element_type=jnp.float32,
                         precision=lax.Precision.HIGHEST)


def _finalize(part, expand, w_o):
    bn = 400
    return pl.pallas_call(
        _fin_body,
        grid=(N // bn,),
        in_specs=[pl.BlockSpec((NC, bn, ROW), lambda i: (0, i, 0)),
                  pl.BlockSpec((ROW, D), lambda i: (0, 0)),
                  pl.BlockSpec((D, D), lambda i: (0, 0))],
        out_specs=pl.BlockSpec((bn, D), lambda i: (i, 0)),
        out_shape=jax.ShapeDtypeStruct((N, D), jnp.float32),
    )(part, expand, w_o)


def kernel(x, edge_index, edge_type, W_q, W_k, W_v, W_o, rel_emb):
    src = edge_index[0].astype(jnp.int32)
    dst = edge_index[1].astype(jnp.int32)
    et = edge_type.astype(jnp.int32)

    w_cat = jnp.concatenate([W_q * (1.0 / jnp.sqrt(jnp.float32(DH))),
                             W_k, W_v], axis=1)
    q_all, kv_x = _project(x, w_cat)
    kv_r = _project_rel(rel_emb, w_cat[:, D:])

    idx3 = jnp.stack([src, dst, et]).reshape(3, NW * CHUNKS, C)
    idx3 = idx3.transpose(1, 0, 2)

    part = _edge_pass(q_all, kv_x, kv_r, idx3)

    expand = jnp.zeros((ROW, D), jnp.float32)
    for h in range(H):
        expand = expand.at[D + LANE_OF[h], h * DH:(h + 1) * DH].set(1.0)

    return _finalize(part, expand, W_o)


# speedup vs baseline: 1.6052x; 1.1091x over previous
import jax
import jax.numpy as jnp
import numpy as np
from jax import lax
from jax.experimental import pallas as pl
from jax.experimental.pallas import tpu as pltpu
from jax.experimental.pallas import tpu_sc as plsc

N = 10000
E = 320000
D = 128
H = 8
DH = 16
N_ACC = 10080
ROW = D + DH
R = 200

NC = 2
NS = 16
NW = NC * NS
E_PER_W = E // NW
C = 40
CHUNKS = E_PER_W // C
ROWS_PER_TILE = N_ACC // NS
WBR = 30

LANE_OF = (0, 8, 4, 12, 2, 10, 6, 14)

_GDN = lax.GatherDimensionNumbers(
    offset_dims=(), collapsed_slice_dims=(0,), start_index_map=(0,))


def _shuf(x, idx):
    return lax.gather(x, idx[:, None], _GDN, (1,),
                      mode=lax.GatherScatterMode.PROMISE_IN_BOUNDS)


def _edge_kernel(q_hbm, kvx_hbm, kvr_hbm, ei_hbm, et_hbm, out_hbm,
                 ix, wrow, qv, kv, acc, kvr_spm, sems):
    cid = lax.axis_index("c")
    sid = lax.axis_index("s")
    wid = cid * NS + sid
    ebase = wid * E_PER_W

    @pl.when(sid == 0)
    def _stage_rel():
        pltpu.sync_copy(kvr_hbm, kvr_spm)

    def _zrow(i, _):
        for j in range(ROW // 16):
            wrow[i, pl.ds(j * 16, 16)] = jnp.zeros((16,), jnp.float32)
        return 0
    lax.fori_loop(0, C, _zrow, 0)
    row0 = sid * ROWS_PER_TILE
    for b in range(ROWS_PER_TILE // WBR):
        pltpu.sync_copy(wrow.at[pl.ds(0, WBR)],
                        acc.at[pl.ds(row0 + b * WBR, WBR)])
    plsc.subcore_barrier()

    lanes = lax.iota(jnp.int32, 16)
    px = {h: lanes ^ h for h in (8, 4, 2, 1)}
    msk = {h: (lanes & h) == 0 for h in (8, 4, 2)}
    bcast = [jnp.full((16,), LANE_OF[h], jnp.int32) for h in range(H)]

    def _merge(a, b, half):
        a2 = a + _shuf(a, px[half])
        b2 = b + _shuf(b, px[half])
        return jnp.where(msk[half], a2, _shuf(b2, px[half]))

    def _load_idx(c, p):
        off = ebase + c * C
        c1 = pltpu.async_copy(ei_hbm.at[0, pl.ds(off, C)], ix.at[p, 0],
                              sems.at[2])
        c2 = pltpu.async_copy(ei_hbm.at[1, pl.ds(off, C)], ix.at[p, 1],
                              sems.at[2])
        c3 = pltpu.async_copy(et_hbm.at[pl.ds(off, C)], ix.at[p, 2],
                              sems.at[2])
        c1.wait()
        c2.wait()
        c3.wait()

    def _issue(p):
        pltpu.async_copy(q_hbm.at[ix.at[p, 1]], qv.at[p], sems.at[p])
        pltpu.async_copy(kvx_hbm.at[ix.at[p, 0]], kv.at[p], sems.at[p])

    def _wait_gathers(p):
        pltpu.make_async_copy(q_hbm.at[ix.at[p, 1]], qv.at[p],
                              sems.at[p]).wait()
        pltpu.make_async_copy(kvx_hbm.at[ix.at[p, 0]], kv.at[p],
                              sems.at[p]).wait()

    def _compute_scatter(p):
        @plsc.parallel_loop(0, C, 1, unroll=2)
        def _edge(e):
            u = [qv[p, e, pl.ds(h * 16, 16)] * kv[p, e, pl.ds(h * 16, 16)]
                 for h in range(H)]
            v1 = [_merge(u[2 * i], u[2 * i + 1], 8) for i in range(4)]
            v2 = [_merge(v1[2 * i], v1[2 * i + 1], 4) for i in range(2)]
            r = _merge(v2[0], v2[1], 2)
            r = r + _shuf(r, px[1])
            pv = jnp.exp(r)
            for h in range(H):
                ph = _shuf(pv, bcast[h])
                wrow[e, pl.ds(h * 16, 16)] = ph * kv[p, e, pl.ds(D + h * 16, 16)]
            wrow[e, pl.ds(D, 16)] = pv
        pltpu.async_copy(wrow, acc.at[ix.at[p, 1]], sems.at[3], add=True)

    def _wait_scatter():
        pltpu.make_async_copy(wrow, acc.at[ix.at[0, 1]], sems.at[3]).wait()

    def _body(i, _):
        c0 = 2 * i
        _wait_gathers(0)
        ga = pltpu.async_copy(kvr_spm.at[ix.at[0, 2]], kv.at[0], sems.at[0],
                              add=True)
        _wait_scatter()
        _load_idx(c0 + 1, 1)
        _issue(1)
        ga.wait()
        _compute_scatter(0)
        _wait_gathers(1)
        gb = pltpu.async_copy(kvr_spm.at[ix.at[1, 2]], kv.at[1], sems.at[1],
                              add=True)
        _wait_scatter()
        c2 = lax.min(c0 + 2, CHUNKS - 1)
        _load_idx(c2, 0)
        _issue(0)
        gb.wait()
        _compute_scatter(1)
        return 0

    _load_idx(0, 0)
    _issue(0)
    pltpu.async_copy(wrow, acc.at[ix.at[0, 1]], sems.at[3], add=True)
    lax.fori_loop(0, CHUNKS // 2, _body, 0)
    _wait_gathers(0)
    _wait_scatter()

    plsc.subcore_barrier()
    for b in range(ROWS_PER_TILE // WBR):
        r0 = row0 + b * WBR
        pltpu.sync_copy(acc.at[pl.ds(r0, WBR)], wrow.at[pl.ds(0, WBR)])
        pltpu.sync_copy(wrow.at[pl.ds(0, WBR)],
                        out_hbm.at[cid, pl.ds(r0, WBR)])


_edge_pass = pl.kernel(
    _edge_kernel,
    out_type=jax.ShapeDtypeStruct((NC, N_ACC, ROW), jnp.float32),
    mesh=plsc.VectorSubcoreMesh(core_axis_name="c", subcore_axis_name="s"),
    compiler_params=pltpu.CompilerParams(needs_layout_passes=False,
                                         use_tc_tiling_on_sc=False),
    scratch_types=[
        pltpu.VMEM((2, 3, C), jnp.int32),
        pltpu.VMEM((C, ROW), jnp.float32),
        pltpu.VMEM((2, C, D), jnp.float32),
        pltpu.VMEM((2, C, 2 * D), jnp.float32),
        pltpu.VMEM_SHARED((N_ACC, ROW), jnp.float32),
        pltpu.VMEM_SHARED((R, 2 * D), jnp.float32),
        pltpu.SemaphoreType.DMA((4,)),
    ],
)


def _proj_body(x_ref, wq_ref, wk_ref, wv_ref, q_ref, kv_ref):
    xb = x_ref[...]
    q_ref[...] = jnp.dot(xb, wq_ref[...], preferred_element_type=jnp.float32,
                         precision=lax.Precision.HIGHEST) * (1.0 / 4.0)
    kv_ref[:, :D] = jnp.dot(xb, wk_ref[...],
                            preferred_element_type=jnp.float32,
                            precision=lax.Precision.HIGHEST)
    kv_ref[:, D:] = jnp.dot(xb, wv_ref[...],
                            preferred_element_type=jnp.float32,
                            precision=lax.Precision.HIGHEST)


def _project(x, w_q, w_k, w_v):
    bn = 400
    wspec = pl.BlockSpec((D, D), lambda i: (0, 0))
    return pl.pallas_call(
        _proj_body,
        grid=(N // bn,),
        in_specs=[pl.BlockSpec((bn, D), lambda i: (i, 0)),
                  wspec, wspec, wspec],
        out_specs=[pl.BlockSpec((bn, D), lambda i: (i, 0)),
                   pl.BlockSpec((bn, 2 * D), lambda i: (i, 0))],
        out_shape=[jax.ShapeDtypeStruct((N, D), jnp.float32),
                   jax.ShapeDtypeStruct((N, 2 * D), jnp.float32)],
    )(x, w_q, w_k, w_v)


def _rel_body(r_ref, wk_ref, wv_ref, o_ref):
    rb = r_ref[...]
    o_ref[:, :D] = jnp.dot(rb, wk_ref[...],
                           preferred_element_type=jnp.float32,
                           precision=lax.Precision.HIGHEST)
    o_ref[:, D:] = jnp.dot(rb, wv_ref[...],
                           preferred_element_type=jnp.float32,
                           precision=lax.Precision.HIGHEST)


def _project_rel(rel_emb, w_k, w_v):
    nr = rel_emb.shape[0]
    return pl.pallas_call(
        _rel_body,
        out_shape=jax.ShapeDtypeStruct((nr, 2 * D), jnp.float32),
    )(rel_emb, w_k, w_v)


def _fin_body(a_ref, ex_ref, wo_ref, o_ref):
    a = a_ref[0] + a_ref[1]
    den = jnp.dot(a, ex_ref[...], preferred_element_type=jnp.float32,
                  precision=lax.Precision.HIGHEST)
    agg = a[:, :D] / (den + 1e-16)
    o_ref[...] = jnp.dot(agg, wo_ref[...], preferred_element_type=jnp.float32,
                         precision=lax.Precision.HIGHEST)


def _finalize(part, expand, w_o):
    bn = 400
    return pl.pallas_call(
        _fin_body,
        grid=(N // bn,),
        in_specs=[pl.BlockSpec((NC, bn, ROW), lambda i: (0, i, 0)),
                  pl.BlockSpec((ROW, D), lambda i: (0, 0)),
                  pl.BlockSpec((D, D), lambda i: (0, 0))],
        out_specs=pl.BlockSpec((bn, D), lambda i: (i, 0)),
        out_shape=jax.ShapeDtypeStruct((N, D), jnp.float32),
    )(part, expand, w_o)


_EXPAND = np.zeros((ROW, D), np.float32)
for _h in range(H):
    _EXPAND[D + LANE_OF[_h], _h * DH:(_h + 1) * DH] = 1.0


def kernel(x, edge_index, edge_type, W_q, W_k, W_v, W_o, rel_emb):
    ei = edge_index.astype(jnp.int32)
    et = edge_type.astype(jnp.int32)

    q_all, kv_x = _project(x, W_q, W_k, W_v)
    kv_r = _project_rel(rel_emb, W_k, W_v)

    part = _edge_pass(q_all, kv_x, kv_r, ei, et)

    return _finalize(part, jnp.asarray(_EXPAND), W_o)


# speedup vs baseline: 1.6163x; 1.0069x over previous
import jax
import jax.numpy as jnp
import numpy as np
from jax import lax
from jax.experimental import pallas as pl
from jax.experimental.pallas import tpu as pltpu
from jax.experimental.pallas import tpu_sc as plsc

N = 10000
E = 320000
D = 128
H = 8
DH = 16
N_ACC = 10080
ROW = D + DH
R = 200

NC = 2
NS = 16
NW = NC * NS
E_PER_W = E // NW
C = 40
CHUNKS = E_PER_W // C
ROWS_PER_TILE = N_ACC // NS
WBR = 30

LANE_OF = (0, 8, 4, 12, 2, 10, 6, 14)

_GDN = lax.GatherDimensionNumbers(
    offset_dims=(), collapsed_slice_dims=(0,), start_index_map=(0,))


def _shuf(x, idx):
    return lax.gather(x, idx[:, None], _GDN, (1,),
                      mode=lax.GatherScatterMode.PROMISE_IN_BOUNDS)


def _edge_kernel(q_hbm, kvx_hbm, kvr_hbm, ei_hbm, et_hbm, out_hbm,
                 ix, wrow, qv, kv, acc, kvr_spm, sems):
    cid = lax.axis_index("c")
    sid = lax.axis_index("s")
    wid = cid * NS + sid
    ebase = wid * E_PER_W

    @pl.when(sid == 0)
    def _stage_rel():
        pltpu.sync_copy(kvr_hbm, kvr_spm)

    def _zrow(i, _):
        for j in range(ROW // 16):
            wrow[i, pl.ds(j * 16, 16)] = jnp.zeros((16,), jnp.float32)
        return 0
    lax.fori_loop(0, C, _zrow, 0)
    row0 = sid * ROWS_PER_TILE
    for b in range(ROWS_PER_TILE // WBR):
        pltpu.sync_copy(wrow.at[pl.ds(0, WBR)],
                        acc.at[pl.ds(row0 + b * WBR, WBR)])
    plsc.subcore_barrier()

    lanes = lax.iota(jnp.int32, 16)
    px = {h: lanes ^ h for h in (8, 4, 2, 1)}
    msk = {h: (lanes & h) == 0 for h in (8, 4, 2)}
    bcast = [jnp.full((16,), LANE_OF[h], jnp.int32) for h in range(H)]

    def _merge(a, b, half):
        a2 = a + _shuf(a, px[half])
        b2 = b + _shuf(b, px[half])
        return jnp.where(msk[half], a2, _shuf(b2, px[half]))

    def _load_idx(c, p):
        off = ebase + c * C
        c1 = pltpu.async_copy(ei_hbm.at[0, pl.ds(off, C)], ix.at[p, 0],
                              sems.at[2])
        c2 = pltpu.async_copy(ei_hbm.at[1, pl.ds(off, C)], ix.at[p, 1],
                              sems.at[2])
        c3 = pltpu.async_copy(et_hbm.at[pl.ds(off, C)], ix.at[p, 2],
                              sems.at[2])
        c1.wait()
        c2.wait()
        c3.wait()

    def _issue(p):
        pltpu.async_copy(q_hbm.at[ix.at[p, 1]], qv.at[p], sems.at[p])
        pltpu.async_copy(kvx_hbm.at[ix.at[p, 0]], kv.at[p], sems.at[p])

    def _wait_gathers(p):
        pltpu.make_async_copy(q_hbm.at[ix.at[p, 1]], qv.at[p],
                              sems.at[p]).wait()
        pltpu.make_async_copy(kvx_hbm.at[ix.at[p, 0]], kv.at[p],
                              sems.at[p]).wait()

    def _compute_scatter(p):
        @plsc.parallel_loop(0, C, 1, unroll=2)
        def _edge(e):
            u = [qv[p, e, pl.ds(h * 16, 16)] * kv[p, e, pl.ds(h * 16, 16)]
                 for h in range(H)]
            v1 = [_merge(u[2 * i], u[2 * i + 1], 8) for i in range(4)]
            v2 = [_merge(v1[2 * i], v1[2 * i + 1], 4) for i in range(2)]
            r = _merge(v2[0], v2[1], 2)
            r = r + _shuf(r, px[1])
            pv = jnp.exp(r)
            for h in range(H):
                ph = _shuf(pv, bcast[h])
                wrow[e, pl.ds(h * 16, 16)] = ph * kv[p, e, pl.ds(D + h * 16, 16)]
            wrow[e, pl.ds(D, 16)] = pv
        pltpu.async_copy(wrow, acc.at[ix.at[p, 1]], sems.at[3], add=True)

    def _wait_scatter():
        pltpu.make_async_copy(wrow, acc.at[ix.at[0, 1]], sems.at[3]).wait()

    def _body(i, _):
        c0 = 2 * i
        _wait_gathers(0)
        ga = pltpu.async_copy(kvr_spm.at[ix.at[0, 2]], kv.at[0], sems.at[0],
                              add=True)
        _wait_scatter()
        _load_idx(c0 + 1, 1)
        _issue(1)
        ga.wait()
        _compute_scatter(0)
        _wait_gathers(1)
        gb = pltpu.async_copy(kvr_spm.at[ix.at[1, 2]], kv.at[1], sems.at[1],
                              add=True)
        _wait_scatter()
        c2 = lax.min(c0 + 2, CHUNKS - 1)
        _load_idx(c2, 0)
        _issue(0)
        gb.wait()
        _compute_scatter(1)
        return 0

    _load_idx(0, 0)
    _issue(0)
    pltpu.async_copy(wrow, acc.at[ix.at[0, 1]], sems.at[3], add=True)
    lax.fori_loop(0, CHUNKS // 2, _body, 0)
    _wait_gathers(0)
    _wait_scatter()

    plsc.subcore_barrier()
    for b in range(ROWS_PER_TILE // WBR):
        r0 = row0 + b * WBR
        pltpu.sync_copy(acc.at[pl.ds(r0, WBR)], wrow.at[pl.ds(0, WBR)])
        pltpu.sync_copy(wrow.at[pl.ds(0, WBR)],
                        out_hbm.at[cid, pl.ds(r0, WBR)])


_edge_pass = pl.kernel(
    _edge_kernel,
    out_type=jax.ShapeDtypeStruct((NC, N_ACC, ROW), jnp.float32),
    mesh=plsc.VectorSubcoreMesh(core_axis_name="c", subcore_axis_name="s"),
    compiler_params=pltpu.CompilerParams(needs_layout_passes=False,
                                         use_tc_tiling_on_sc=False),
    scratch_types=[
        pltpu.VMEM((2, 3, C), jnp.int32),
        pltpu.VMEM((C, ROW), jnp.float32),
        pltpu.VMEM((2, C, D), jnp.float32),
        pltpu.VMEM((2, C, 2 * D), jnp.float32),
        pltpu.VMEM_SHARED((N_ACC, ROW), jnp.float32),
        pltpu.VMEM_SHARED((R, 2 * D), jnp.float32),
        pltpu.SemaphoreType.DMA((4,)),
    ],
)


def _proj_body(x_ref, wq_ref, wk_ref, wv_ref, q_ref, kv_ref):
    xb = x_ref[...]
    q_ref[...] = jnp.dot(xb, wq_ref[...], preferred_element_type=jnp.float32) * (1.0 / 4.0)
    kv_ref[:, :D] = jnp.dot(xb, wk_ref[...],
                            preferred_element_type=jnp.float32)
    kv_ref[:, D:] = jnp.dot(xb, wv_ref[...],
                            preferred_element_type=jnp.float32)


def _project(x, w_q, w_k, w_v):
    bn = 400
    wspec = pl.BlockSpec((D, D), lambda i: (0, 0))
    return pl.pallas_call(
        _proj_body,
        grid=(N // bn,),
        in_specs=[pl.BlockSpec((bn, D), lambda i: (i, 0)),
                  wspec, wspec, wspec],
        out_specs=[pl.BlockSpec((bn, D), lambda i: (i, 0)),
                   pl.BlockSpec((bn, 2 * D), lambda i: (i, 0))],
        out_shape=[jax.ShapeDtypeStruct((N, D), jnp.float32),
                   jax.ShapeDtypeStruct((N, 2 * D), jnp.float32)],
    )(x, w_q, w_k, w_v)


def _rel_body(r_ref, wk_ref, wv_ref, o_ref):
    rb = r_ref[...]
    o_ref[:, :D] = jnp.dot(rb, wk_ref[...],
                           preferred_element_type=jnp.float32)
    o_ref[:, D:] = jnp.dot(rb, wv_ref[...],
                           preferred_element_type=jnp.float32)


def _project_rel(rel_emb, w_k, w_v):
    nr = rel_emb.shape[0]
    return pl.pallas_call(
        _rel_body,
        out_shape=jax.ShapeDtypeStruct((nr, 2 * D), jnp.float32),
    )(rel_emb, w_k, w_v)


def _fin_body(a_ref, ex_ref, wo_ref, o_ref):
    a = a_ref[0] + a_ref[1]
    den = jnp.dot(a, ex_ref[...], preferred_element_type=jnp.float32,
                  precision=lax.Precision.HIGHEST)
    agg = a[:, :D] / (den + 1e-16)
    o_ref[...] = jnp.dot(agg, wo_ref[...], preferred_element_type=jnp.float32,
                         precision=lax.Precision.HIGHEST)


def _finalize(part, expand, w_o):
    bn = 400
    return pl.pallas_call(
        _fin_body,
        grid=(N // bn,),
        in_specs=[pl.BlockSpec((NC, bn, ROW), lambda i: (0, i, 0)),
                  pl.BlockSpec((ROW, D), lambda i: (0, 0)),
                  pl.BlockSpec((D, D), lambda i: (0, 0))],
        out_specs=pl.BlockSpec((bn, D), lambda i: (i, 0)),
        out_shape=jax.ShapeDtypeStruct((N, D), jnp.float32),
    )(part, expand, w_o)


_EXPAND = np.zeros((ROW, D), np.float32)
for _h in range(H):
    _EXPAND[D + LANE_OF[_h], _h * DH:(_h + 1) * DH] = 1.0


def kernel(x, edge_index, edge_type, W_q, W_k, W_v, W_o, rel_emb):
    ei = edge_index.astype(jnp.int32)
    et = edge_type.astype(jnp.int32)

    q_all, kv_x = _project(x, W_q, W_k, W_v)
    kv_r = _project_rel(rel_emb, W_k, W_v)

    part = _edge_pass(q_all, kv_x, kv_r, ei, et)

    return _finalize(part, jnp.asarray(_EXPAND), W_o)
